# baseline (speedup 1.0000x reference)
import functools

import jax
import jax.numpy as jnp
from jax import lax
from jax.experimental import pallas as pl
from jax.experimental.pallas import tpu as pltpu
from jax.experimental.pallas import tpu_sc as plsc

NN = 10000
NE = 10000
NNZ = 320000
C = 128
NC = 2
NS = 16
NW = NC * NS
EPW = NNZ // NW
CH = 80
NCH = EPW // CH
ACC_ROWS = 10240
ZR = ACC_ROWS // NS


def _phase_body(with_db, *refs):
    if with_db:
        (tab_hbm, gi_hbm, si_hbm, hew_hbm, acc_out, dp_out, bp_out,
         gi_v, si_v, rows0_v, rows1_v, hew_v, d_v, b_v, acc_sh,
         gsem0, gsem1, ssem0, ssem1) = refs
    else:
        (tab_hbm, gi_hbm, si_hbm, acc_out,
         gi_v, si_v, rows0_v, rows1_v, acc_sh,
         gsem0, gsem1, ssem0, ssem1) = refs
    cid = lax.axis_index("c")
    sid = lax.axis_index("s")
    wid = sid * NC + cid

    pltpu.sync_copy(gi_hbm.at[wid], gi_v)
    pltpu.sync_copy(si_hbm.at[wid], si_v)
    if with_db:
        pltpu.sync_copy(hew_hbm, hew_v)
        z16 = jnp.zeros((16,), jnp.float32)

        def zdb(i, carry):
            d_v[pl.ds(i * 16, 16)] = z16
            b_v[pl.ds(i * 16, 16)] = z16
            return carry

        lax.fori_loop(0, NN // 16, zdb, 0)

    ones16 = jnp.ones((16,), jnp.float32)

    def degree_work(c):
        if with_db:
            for g in range(CH // 16):
                s16 = gi_v[c, pl.ds(g * 16, 16)]
                d16 = si_v[c, pl.ds(g * 16, 16)]
                w16 = plsc.load_gather(hew_v, [d16])
                plsc.addupdate_scatter(d_v, [s16], w16)
                plsc.addupdate_scatter(b_v, [d16], ones16)

    z32 = jnp.zeros((32,), jnp.bfloat16)

    def zrow(i, carry):
        rows0_v[i // 4, pl.ds((i % 4) * 32, 32)] = z32
        return carry

    lax.fori_loop(0, CH * 4, zrow, 0)
    zbase = pl.multiple_of(sid * ZR, 8)
    for k in range(ZR // CH):
        pltpu.sync_copy(rows0_v, acc_sh.at[pl.ds(zbase + k * CH, CH)])
    plsc.subcore_barrier()

    def gather(c, rows_v, sem):
        pltpu.async_copy(tab_hbm.at[gi_v.at[c]], rows_v, sem)

    def gather_wait(c, rows_v, sem):
        pltpu.make_async_copy(tab_hbm.at[gi_v.at[c]], rows_v, sem).wait()

    def scatter(c, rows_v, sem):
        pltpu.async_copy(rows_v, acc_sh.at[si_v.at[c]], sem, add=True)

    def scatter_wait(c, rows_v, sem):
        pltpu.make_async_copy(rows_v, acc_sh.at[si_v.at[c]], sem).wait()

    gather(0, rows0_v, gsem0)

    def pair(i, carry):
        c0 = 2 * i
        gather_wait(c0, rows0_v, gsem0)

        @pl.when(i > 0)
        def _():
            scatter_wait(c0 - 1, rows1_v, ssem1)

        gather(c0 + 1, rows1_v, gsem1)
        scatter(c0, rows0_v, ssem0)
        degree_work(c0)
        gather_wait(c0 + 1, rows1_v, gsem1)
        scatter_wait(c0, rows0_v, ssem0)

        @pl.when(c0 + 2 < NCH)
        def _():
            gather(c0 + 2, rows0_v, gsem0)

        scatter(c0 + 1, rows1_v, ssem1)
        degree_work(c0 + 1)
        return carry

    lax.fori_loop(0, NCH // 2, pair, 0)
    c_last = NCH - 1
    gather_wait(c_last, rows0_v, gsem0)
    scatter_wait(c_last - 1, rows1_v, ssem1)
    scatter(c_last, rows0_v, ssem0)
    degree_work(c_last)
    scatter_wait(c_last, rows0_v, ssem0)
    plsc.subcore_barrier()

    pltpu.sync_copy(acc_sh.at[pl.ds(zbase, ZR)],
                    acc_out.at[cid, pl.ds(zbase, ZR)])
    if with_db:
        obase = pl.multiple_of(wid * NN, 8)
        pltpu.sync_copy(d_v, dp_out.at[pl.ds(obase, NN)])
        pltpu.sync_copy(b_v, bp_out.at[pl.ds(obase, NN)])


_MESH = plsc.VectorSubcoreMesh(core_axis_name="c", subcore_axis_name="s",
                               num_cores=NC, num_subcores=NS)
_SC_PARAMS = pltpu.CompilerParams(needs_layout_passes=False,
                                  use_tc_tiling_on_sc=False)

_phase1 = pl.kernel(
    functools.partial(_phase_body, True),
    out_type=(
        jax.ShapeDtypeStruct((NC, ACC_ROWS, C), jnp.bfloat16),
        jax.ShapeDtypeStruct((NW * NN,), jnp.float32),
        jax.ShapeDtypeStruct((NW * NE,), jnp.float32),
    ),
    mesh=_MESH,
    compiler_params=_SC_PARAMS,
    scratch_types=(
        pltpu.VMEM((NCH, CH), jnp.int32),
        pltpu.VMEM((NCH, CH), jnp.int32),
        pltpu.VMEM((CH, C), jnp.bfloat16),
        pltpu.VMEM((CH, C), jnp.bfloat16),
        pltpu.VMEM((NE,), jnp.float32),
        pltpu.VMEM((NN,), jnp.float32),
        pltpu.VMEM((NE,), jnp.float32),
        pltpu.VMEM_SHARED((ACC_ROWS, C), jnp.bfloat16),
        pltpu.SemaphoreType.DMA,
        pltpu.SemaphoreType.DMA,
        pltpu.SemaphoreType.DMA,
        pltpu.SemaphoreType.DMA,
    ),
)

_phase2 = pl.kernel(
    functools.partial(_phase_body, False),
    out_type=jax.ShapeDtypeStruct((NC, ACC_ROWS, C), jnp.bfloat16),
    mesh=_MESH,
    compiler_params=_SC_PARAMS,
    scratch_types=(
        pltpu.VMEM((NCH, CH), jnp.int32),
        pltpu.VMEM((NCH, CH), jnp.int32),
        pltpu.VMEM((CH, C), jnp.bfloat16),
        pltpu.VMEM((CH, C), jnp.bfloat16),
        pltpu.VMEM_SHARED((ACC_ROWS, C), jnp.bfloat16),
        pltpu.SemaphoreType.DMA,
        pltpu.SemaphoreType.DMA,
        pltpu.SemaphoreType.DMA,
        pltpu.SemaphoreType.DMA,
    ),
)


def _combine_edges_body(ep_ref, bp_ref, out_ref):
    bsum = jnp.sum(bp_ref[...], axis=0)
    binv = jnp.where(bsum > 0, 1.0 / bsum, 0.0)
    esum = (ep_ref[0, :NE].astype(jnp.float32)
            + ep_ref[1, :NE].astype(jnp.float32))
    out_ref[...] = (esum * binv[:, None]).astype(jnp.bfloat16)


def _finish_body(np_ref, dp_ref, w_ref, b_ref, out_ref):
    dsum = jnp.sum(dp_ref[...], axis=0)
    dinv = jnp.where(dsum > 0, 1.0 / dsum, 0.0)
    t = (np_ref[0, :NN].astype(jnp.float32)
         + np_ref[1, :NN].astype(jnp.float32)) * dinv[:, None]
    out_ref[...] = (jnp.dot(t, w_ref[...], preferred_element_type=jnp.float32)
                    + b_ref[...])


def kernel(x, HE, HEW, W, b):
    src = HE[0]
    dst = HE[1]
    src_w = src.reshape(NW, NCH, CH)
    dst_w = dst.reshape(NW, NCH, CH)
    xb = x.astype(jnp.bfloat16)

    ep, dp, bp = _phase1(xb, src_w, dst_w, HEW)
    dp = dp.reshape(NW, NN)
    bp = bp.reshape(NW, NE)

    ef = pl.pallas_call(
        _combine_edges_body,
        out_shape=jax.ShapeDtypeStruct((NE, C), jnp.bfloat16),
    )(ep, bp)

    npar = _phase2(ef, dst_w, src_w)

    out = pl.pallas_call(
        _finish_body,
        out_shape=jax.ShapeDtypeStruct((NN, C), jnp.float32),
    )(npar, dp, W, b.reshape(1, C))
    return out


# speedup vs baseline: 1.1556x; 1.1556x over previous
import functools

import jax
import jax.numpy as jnp
from jax import lax
from jax.experimental import pallas as pl
from jax.experimental.pallas import tpu as pltpu
from jax.experimental.pallas import tpu_sc as plsc

NN = 10000
NE = 10000
NNZ = 320000
C = 128
NC = 2
NS = 16
NW = NC * NS
EPW = NNZ // NW
CH = 80
NCH = EPW // CH
CHP = 125
NCHP = EPW // CHP
ACC_ROWS = 10240
ZR = ACC_ROWS // NS


def _degree_body(gi_hbm, si_hbm, hew_hbm, dp_out, bp_out,
                 gi_v, si_v, hew_v, d_v, b_v):
    cid = lax.axis_index("c")
    sid = lax.axis_index("s")
    wid = sid * NC + cid

    pltpu.sync_copy(gi_hbm.at[wid], gi_v)
    pltpu.sync_copy(si_hbm.at[wid], si_v)
    pltpu.sync_copy(hew_hbm, hew_v)

    z16 = jnp.zeros((16,), jnp.float32)

    def zdb(i, carry):
        d_v[pl.ds(i * 16, 16)] = z16
        b_v[pl.ds(i * 16, 16)] = z16
        return carry

    lax.fori_loop(0, NN // 16, zdb, 0)

    ones16 = jnp.ones((16,), jnp.float32)

    def chunk(c, carry):
        for g in range(CH // 16):
            s16 = gi_v[c, pl.ds(g * 16, 16)]
            d16 = si_v[c, pl.ds(g * 16, 16)]
            w16 = plsc.load_gather(hew_v, [d16])
            plsc.addupdate_scatter(d_v, [s16], w16)
            plsc.addupdate_scatter(b_v, [d16], ones16)
        return carry

    lax.fori_loop(0, NCH, chunk, 0)

    obase = pl.multiple_of(wid * NN, 8)
    pltpu.sync_copy(d_v, dp_out.at[pl.ds(obase, NN)])
    pltpu.sync_copy(b_v, bp_out.at[pl.ds(obase, NN)])


def _phase_body(tab_hbm, gi_hbm, si_hbm, acc_out,
                gi_v, si_v, rows0_v, rows1_v, acc_sh,
                gsem0, gsem1, ssem0, ssem1):
    cid = lax.axis_index("c")
    sid = lax.axis_index("s")
    wid = sid * NC + cid

    pltpu.sync_copy(gi_hbm.at[wid], gi_v)
    pltpu.sync_copy(si_hbm.at[wid], si_v)

    z32 = jnp.zeros((32,), jnp.bfloat16)

    def zrow(i, carry):
        rows0_v[i // 4, pl.ds((i % 4) * 32, 32)] = z32
        return carry

    lax.fori_loop(0, CHP * 4, zrow, 0)
    zbase = pl.multiple_of(sid * ZR, 8)
    for k in range(ZR // CH):
        pltpu.sync_copy(rows0_v.at[pl.ds(0, CH)],
                        acc_sh.at[pl.ds(zbase + k * CH, CH)])
    plsc.subcore_barrier()

    def gather(c, rows_v, sem):
        pltpu.async_copy(tab_hbm.at[gi_v.at[c]], rows_v, sem)

    def gather_wait(c, rows_v, sem):
        pltpu.make_async_copy(tab_hbm.at[gi_v.at[c]], rows_v, sem).wait()

    def scatter(c, rows_v, sem):
        pltpu.async_copy(rows_v, acc_sh.at[si_v.at[c]], sem, add=True)

    def scatter_wait(c, rows_v, sem):
        pltpu.make_async_copy(rows_v, acc_sh.at[si_v.at[c]], sem).wait()

    gather(0, rows0_v, gsem0)

    def pair(i, carry):
        c0 = 2 * i
        gather_wait(c0, rows0_v, gsem0)

        @pl.when(i > 0)
        def _():
            scatter_wait(c0 - 1, rows1_v, ssem1)

        gather(c0 + 1, rows1_v, gsem1)
        scatter(c0, rows0_v, ssem0)
        gather_wait(c0 + 1, rows1_v, gsem1)
        scatter_wait(c0, rows0_v, ssem0)

        @pl.when(c0 + 2 < NCHP)
        def _():
            gather(c0 + 2, rows0_v, gsem0)

        scatter(c0 + 1, rows1_v, ssem1)
        return carry

    lax.fori_loop(0, NCHP // 2, pair, 0)
    scatter_wait(NCHP - 1, rows1_v, ssem1)
    plsc.subcore_barrier()

    pltpu.sync_copy(acc_sh.at[pl.ds(zbase, ZR)],
                    acc_out.at[cid, pl.ds(zbase, ZR)])


_MESH = plsc.VectorSubcoreMesh(core_axis_name="c", subcore_axis_name="s",
                               num_cores=NC, num_subcores=NS)
_SC_PARAMS = pltpu.CompilerParams(needs_layout_passes=False,
                                  use_tc_tiling_on_sc=False)

_degree = pl.kernel(
    _degree_body,
    out_type=(
        jax.ShapeDtypeStruct((NW * NN,), jnp.float32),
        jax.ShapeDtypeStruct((NW * NE,), jnp.float32),
    ),
    mesh=_MESH,
    compiler_params=_SC_PARAMS,
    scratch_types=(
        pltpu.VMEM((NCH, CH), jnp.int32),
        pltpu.VMEM((NCH, CH), jnp.int32),
        pltpu.VMEM((NE,), jnp.float32),
        pltpu.VMEM((NN,), jnp.float32),
        pltpu.VMEM((NE,), jnp.float32),
    ),
)

_phase = pl.kernel(
    _phase_body,
    out_type=jax.ShapeDtypeStruct((NC, ACC_ROWS, C), jnp.bfloat16),
    mesh=_MESH,
    compiler_params=_SC_PARAMS,
    scratch_types=(
        pltpu.VMEM((NCHP, CHP), jnp.int32),
        pltpu.VMEM((NCHP, CHP), jnp.int32),
        pltpu.VMEM((CHP, C), jnp.bfloat16),
        pltpu.VMEM((CHP, C), jnp.bfloat16),
        pltpu.VMEM_SHARED((ACC_ROWS, C), jnp.bfloat16),
        pltpu.SemaphoreType.DMA,
        pltpu.SemaphoreType.DMA,
        pltpu.SemaphoreType.DMA,
        pltpu.SemaphoreType.DMA,
    ),
)


def _combine_edges_body(ep_ref, bp_ref, out_ref):
    bsum = jnp.sum(bp_ref[...], axis=0)
    binv = jnp.where(bsum > 0, 1.0 / bsum, 0.0)
    esum = (ep_ref[0, :NE].astype(jnp.float32)
            + ep_ref[1, :NE].astype(jnp.float32))
    out_ref[...] = (esum * binv[:, None]).astype(jnp.bfloat16)


def _finish_body(np_ref, dp_ref, w_ref, b_ref, out_ref):
    dsum = jnp.sum(dp_ref[...], axis=0)
    dinv = jnp.where(dsum > 0, 1.0 / dsum, 0.0)
    t = (np_ref[0, :NN].astype(jnp.float32)
         + np_ref[1, :NN].astype(jnp.float32)) * dinv[:, None]
    out_ref[...] = (jnp.dot(t, w_ref[...], preferred_element_type=jnp.float32)
                    + b_ref[...])


def kernel(x, HE, HEW, W, b):
    src = HE[0]
    dst = HE[1]
    src_w = src.reshape(NW, NCH, CH)
    dst_w = dst.reshape(NW, NCH, CH)
    src_p = src.reshape(NW, NCHP, CHP)
    dst_p = dst.reshape(NW, NCHP, CHP)
    xb = x.astype(jnp.bfloat16)

    dp, bp = _degree(src_w, dst_w, HEW)
    dp = dp.reshape(NW, NN)
    bp = bp.reshape(NW, NE)

    ep = _phase(xb, src_p, dst_p)

    ef = pl.pallas_call(
        _combine_edges_body,
        out_shape=jax.ShapeDtypeStruct((NE, C), jnp.bfloat16),
    )(ep, bp)

    npar = _phase(ef, dst_p, src_p)

    out = pl.pallas_call(
        _finish_body,
        out_shape=jax.ShapeDtypeStruct((NN, C), jnp.float32),
    )(npar, dp, W, b.reshape(1, C))
    return out


# speedup vs baseline: 1.4342x; 1.2411x over previous
import functools

import jax
import jax.numpy as jnp
from jax import lax
from jax.experimental import pallas as pl
from jax.experimental.pallas import tpu as pltpu
from jax.experimental.pallas import tpu_sc as plsc

NN = 10000
NE = 10000
NNZ = 320000
C = 128
NC = 2
NS = 16
NW = NC * NS
EPW = NNZ // NW
CH = 80
NCH = EPW // CH
CHP = 125
NCHP = EPW // CHP
ACC_ROWS = 10240
ZR = ACC_ROWS // NS


def _degree_body(gi_hbm, si_hbm, hew_hbm, dp_out, bp_out,
                 gi_v, si_v, hew_v, d_v, b_v):
    cid = lax.axis_index("c")
    sid = lax.axis_index("s")
    wid = sid * NC + cid

    pltpu.sync_copy(gi_hbm.at[wid], gi_v)
    pltpu.sync_copy(si_hbm.at[wid], si_v)
    pltpu.sync_copy(hew_hbm, hew_v)

    z16 = jnp.zeros((16,), jnp.float32)

    def zdb(i, carry):
        d_v[pl.ds(i * 16, 16)] = z16
        b_v[pl.ds(i * 16, 16)] = z16
        return carry

    lax.fori_loop(0, NN // 16, zdb, 0)

    ones16 = jnp.ones((16,), jnp.float32)

    def chunk(c, carry):
        for g in range(CH // 16):
            s16 = gi_v[c, pl.ds(g * 16, 16)]
            d16 = si_v[c, pl.ds(g * 16, 16)]
            w16 = plsc.load_gather(hew_v, [d16])
            plsc.addupdate_scatter(d_v, [s16], w16)
            plsc.addupdate_scatter(b_v, [d16], ones16)
        return carry

    lax.fori_loop(0, NCH, chunk, 0)

    obase = pl.multiple_of(wid * NN, 8)
    pltpu.sync_copy(d_v, dp_out.at[pl.ds(obase, NN)])
    pltpu.sync_copy(b_v, bp_out.at[pl.ds(obase, NN)])


NBUF = 4


def _phase_body(tab_hbm, gi_hbm, si_hbm, acc_out,
                gi_v, si_v, *bufs_and_sems):
    rows = bufs_and_sems[:NBUF]
    acc_sh = bufs_and_sems[NBUF]
    gs = bufs_and_sems[NBUF + 1:2 * NBUF + 1]
    ss = bufs_and_sems[2 * NBUF + 1:]
    cid = lax.axis_index("c")
    sid = lax.axis_index("s")
    wid = sid * NC + cid

    pltpu.sync_copy(gi_hbm.at[wid], gi_v)
    pltpu.sync_copy(si_hbm.at[wid], si_v)

    z32 = jnp.zeros((32,), jnp.bfloat16)

    def zrow(i, carry):
        rows[0][i // 4, pl.ds((i % 4) * 32, 32)] = z32
        return carry

    lax.fori_loop(0, CHP * 4, zrow, 0)
    zbase = pl.multiple_of(sid * ZR, 8)
    for k in range(ZR // CH):
        pltpu.sync_copy(rows[0].at[pl.ds(0, CH)],
                        acc_sh.at[pl.ds(zbase + k * CH, CH)])
    plsc.subcore_barrier()

    def gather(c, rows_v, sem):
        pltpu.async_copy(tab_hbm.at[gi_v.at[c]], rows_v, sem)

    def gather_wait(c, rows_v, sem):
        pltpu.make_async_copy(tab_hbm.at[gi_v.at[c]], rows_v, sem).wait()

    def scatter(c, rows_v, sem):
        pltpu.async_copy(rows_v, acc_sh.at[si_v.at[c]], sem, add=True)

    def scatter_wait(c, rows_v, sem):
        pltpu.make_async_copy(rows_v, acc_sh.at[si_v.at[c]], sem).wait()

    for j in range(NBUF):
        gather(j, rows[j], gs[j])

    def block(i, carry):
        c0 = i * NBUF
        for j in range(NBUF):
            gather_wait(c0 + j, rows[j], gs[j])
            scatter(c0 + j, rows[j], ss[j])
        for j in range(NBUF):
            cn = c0 + NBUF + j

            @pl.when(cn < NCHP)
            def _(j=j, cn=cn):
                scatter_wait(c0 + j, rows[j], ss[j])
                gather(cn, rows[j], gs[j])

        return carry

    lax.fori_loop(0, NCHP // NBUF, block, 0)
    for j in range(NBUF):
        scatter_wait(NCHP - NBUF + j, rows[j], ss[j])
    plsc.subcore_barrier()

    pltpu.sync_copy(acc_sh.at[pl.ds(zbase, ZR)],
                    acc_out.at[cid, pl.ds(zbase, ZR)])


_MESH = plsc.VectorSubcoreMesh(core_axis_name="c", subcore_axis_name="s",
                               num_cores=NC, num_subcores=NS)
_SC_PARAMS = pltpu.CompilerParams(needs_layout_passes=False,
                                  use_tc_tiling_on_sc=False)

_degree = pl.kernel(
    _degree_body,
    out_type=(
        jax.ShapeDtypeStruct((NW * NN,), jnp.float32),
        jax.ShapeDtypeStruct((NW * NE,), jnp.float32),
    ),
    mesh=_MESH,
    compiler_params=_SC_PARAMS,
    scratch_types=(
        pltpu.VMEM((NCH, CH), jnp.int32),
        pltpu.VMEM((NCH, CH), jnp.int32),
        pltpu.VMEM((NE,), jnp.float32),
        pltpu.VMEM((NN,), jnp.float32),
        pltpu.VMEM((NE,), jnp.float32),
    ),
)

_phase = pl.kernel(
    _phase_body,
    out_type=jax.ShapeDtypeStruct((NC, ACC_ROWS, C), jnp.bfloat16),
    mesh=_MESH,
    compiler_params=_SC_PARAMS,
    scratch_types=(
        (pltpu.VMEM((NCHP, CHP), jnp.int32),
         pltpu.VMEM((NCHP, CHP), jnp.int32))
        + tuple(pltpu.VMEM((CHP, C), jnp.bfloat16) for _ in range(NBUF))
        + (pltpu.VMEM_SHARED((ACC_ROWS, C), jnp.bfloat16),)
        + tuple(pltpu.SemaphoreType.DMA for _ in range(2 * NBUF))
    ),
)


def _combine_edges_body(ep_ref, bp_ref, out_ref):
    bsum = jnp.sum(bp_ref[...], axis=0)
    binv = jnp.where(bsum > 0, 1.0 / bsum, 0.0)
    esum = (ep_ref[0, :NE].astype(jnp.float32)
            + ep_ref[1, :NE].astype(jnp.float32))
    out_ref[...] = (esum * binv[:, None]).astype(jnp.bfloat16)


def _finish_body(np_ref, dp_ref, w_ref, b_ref, out_ref):
    dsum = jnp.sum(dp_ref[...], axis=0)
    dinv = jnp.where(dsum > 0, 1.0 / dsum, 0.0)
    t = (np_ref[0, :NN].astype(jnp.float32)
         + np_ref[1, :NN].astype(jnp.float32)) * dinv[:, None]
    out_ref[...] = (jnp.dot(t, w_ref[...], preferred_element_type=jnp.float32)
                    + b_ref[...])


def kernel(x, HE, HEW, W, b):
    src = HE[0]
    dst = HE[1]
    src_w = src.reshape(NW, NCH, CH)
    dst_w = dst.reshape(NW, NCH, CH)
    src_p = src.reshape(NW, NCHP, CHP)
    dst_p = dst.reshape(NW, NCHP, CHP)
    xb = x.astype(jnp.bfloat16)

    dp, bp = _degree(src_w, dst_w, HEW)
    dp = dp.reshape(NW, NN)
    bp = bp.reshape(NW, NE)

    ep = _phase(xb, src_p, dst_p)

    ef = pl.pallas_call(
        _combine_edges_body,
        out_shape=jax.ShapeDtypeStruct((NE, C), jnp.bfloat16),
    )(ep, bp)

    npar = _phase(ef, dst_p, src_p)

    out = pl.pallas_call(
        _finish_body,
        out_shape=jax.ShapeDtypeStruct((NN, C), jnp.float32),
    )(npar, dp, W, b.reshape(1, C))
    return out


# speedup vs baseline: 1.4788x; 1.0311x over previous
import functools

import jax
import jax.numpy as jnp
from jax import lax
from jax.experimental import pallas as pl
from jax.experimental.pallas import tpu as pltpu
from jax.experimental.pallas import tpu_sc as plsc

NN = 10000
NE = 10000
NNZ = 320000
C = 128
NC = 2
NS = 16
NW = NC * NS
EPW = NNZ // NW
CH = 80
NCH = EPW // CH
CHP = 125
NCHP = EPW // CHP
ACC_ROWS = 10240
ZR = ACC_ROWS // NS


def _degree_body(gi_hbm, si_hbm, hew_hbm, dp_out, bp_out,
                 gi_v, si_v, hew_v, d_v, b_v):
    cid = lax.axis_index("c")
    sid = lax.axis_index("s")
    wid = sid * NC + cid

    pltpu.sync_copy(gi_hbm.at[wid], gi_v)
    pltpu.sync_copy(si_hbm.at[wid], si_v)
    pltpu.sync_copy(hew_hbm, hew_v)

    z16 = jnp.zeros((16,), jnp.float32)

    def zdb(i, carry):
        d_v[pl.ds(i * 16, 16)] = z16
        b_v[pl.ds(i * 16, 16)] = z16
        return carry

    lax.fori_loop(0, NN // 16, zdb, 0)

    ones16 = jnp.ones((16,), jnp.float32)

    def chunk(c, carry):
        for g in range(CH // 16):
            s16 = gi_v[c, pl.ds(g * 16, 16)]
            d16 = si_v[c, pl.ds(g * 16, 16)]
            w16 = plsc.load_gather(hew_v, [d16])
            plsc.addupdate_scatter(d_v, [s16], w16)
            plsc.addupdate_scatter(b_v, [d16], ones16)
        return carry

    lax.fori_loop(0, NCH, chunk, 0)

    obase = pl.multiple_of(wid * NN, 8)
    pltpu.sync_copy(d_v, dp_out.at[pl.ds(obase, NN)])
    pltpu.sync_copy(b_v, bp_out.at[pl.ds(obase, NN)])


NBUF = 8


def _phase_body(tab_hbm, gi_hbm, si_hbm, acc_out,
                gi_v, si_v, *bufs_and_sems):
    rows = bufs_and_sems[:NBUF]
    acc_sh = bufs_and_sems[NBUF]
    gs = bufs_and_sems[NBUF + 1:2 * NBUF + 1]
    ss = bufs_and_sems[2 * NBUF + 1:]
    cid = lax.axis_index("c")
    sid = lax.axis_index("s")
    wid = sid * NC + cid

    pltpu.sync_copy(gi_hbm.at[wid], gi_v)
    pltpu.sync_copy(si_hbm.at[wid], si_v)

    z32 = jnp.zeros((32,), jnp.bfloat16)

    def zrow(i, carry):
        rows[0][i // 4, pl.ds((i % 4) * 32, 32)] = z32
        return carry

    lax.fori_loop(0, CHP * 4, zrow, 0)
    zbase = pl.multiple_of(sid * ZR, 8)
    for k in range(ZR // CH):
        pltpu.sync_copy(rows[0].at[pl.ds(0, CH)],
                        acc_sh.at[pl.ds(zbase + k * CH, CH)])
    plsc.subcore_barrier()

    def gather(c, rows_v, sem):
        pltpu.async_copy(tab_hbm.at[gi_v.at[c]], rows_v, sem)

    def gather_wait(c, rows_v, sem):
        pltpu.make_async_copy(tab_hbm.at[gi_v.at[c]], rows_v, sem).wait()

    def scatter(c, rows_v, sem):
        pltpu.async_copy(rows_v, acc_sh.at[si_v.at[c]], sem, add=True)

    def scatter_wait(c, rows_v, sem):
        pltpu.make_async_copy(rows_v, acc_sh.at[si_v.at[c]], sem).wait()

    for j in range(NBUF):
        gather(j, rows[j], gs[j])

    def block(i, carry):
        c0 = i * NBUF
        for j in range(NBUF):
            gather_wait(c0 + j, rows[j], gs[j])
            scatter(c0 + j, rows[j], ss[j])
        for j in range(NBUF):
            cn = c0 + NBUF + j

            @pl.when(cn < NCHP)
            def _(j=j, cn=cn):
                scatter_wait(c0 + j, rows[j], ss[j])
                gather(cn, rows[j], gs[j])

        return carry

    lax.fori_loop(0, NCHP // NBUF, block, 0)
    for j in range(NBUF):
        scatter_wait(NCHP - NBUF + j, rows[j], ss[j])
    plsc.subcore_barrier()

    pltpu.sync_copy(acc_sh.at[pl.ds(zbase, ZR)],
                    acc_out.at[cid, pl.ds(zbase, ZR)])


_MESH = plsc.VectorSubcoreMesh(core_axis_name="c", subcore_axis_name="s",
                               num_cores=NC, num_subcores=NS)
_SC_PARAMS = pltpu.CompilerParams(needs_layout_passes=False,
                                  use_tc_tiling_on_sc=False)

_degree = pl.kernel(
    _degree_body,
    out_type=(
        jax.ShapeDtypeStruct((NW * NN,), jnp.float32),
        jax.ShapeDtypeStruct((NW * NE,), jnp.float32),
    ),
    mesh=_MESH,
    compiler_params=_SC_PARAMS,
    scratch_types=(
        pltpu.VMEM((NCH, CH), jnp.int32),
        pltpu.VMEM((NCH, CH), jnp.int32),
        pltpu.VMEM((NE,), jnp.float32),
        pltpu.VMEM((NN,), jnp.float32),
        pltpu.VMEM((NE,), jnp.float32),
    ),
)

_phase = pl.kernel(
    _phase_body,
    out_type=jax.ShapeDtypeStruct((NC, ACC_ROWS, C), jnp.bfloat16),
    mesh=_MESH,
    compiler_params=_SC_PARAMS,
    scratch_types=(
        (pltpu.VMEM((NCHP, CHP), jnp.int32),
         pltpu.VMEM((NCHP, CHP), jnp.int32))
        + tuple(pltpu.VMEM((CHP, C), jnp.bfloat16) for _ in range(NBUF))
        + (pltpu.VMEM_SHARED((ACC_ROWS, C), jnp.bfloat16),)
        + tuple(pltpu.SemaphoreType.DMA for _ in range(2 * NBUF))
    ),
)


def _combine_edges_body(ep_ref, bp_ref, out_ref):
    bsum = jnp.sum(bp_ref[...], axis=0)
    binv = jnp.where(bsum > 0, 1.0 / bsum, 0.0)
    esum = (ep_ref[0, :NE].astype(jnp.float32)
            + ep_ref[1, :NE].astype(jnp.float32))
    out_ref[...] = (esum * binv[:, None]).astype(jnp.bfloat16)


def _finish_body(np_ref, dp_ref, w_ref, b_ref, out_ref):
    dsum = jnp.sum(dp_ref[...], axis=0)
    dinv = jnp.where(dsum > 0, 1.0 / dsum, 0.0)
    t = (np_ref[0, :NN].astype(jnp.float32)
         + np_ref[1, :NN].astype(jnp.float32)) * dinv[:, None]
    out_ref[...] = (jnp.dot(t, w_ref[...], preferred_element_type=jnp.float32)
                    + b_ref[...])


def kernel(x, HE, HEW, W, b):
    src = HE[0]
    dst = HE[1]
    src_w = src.reshape(NW, NCH, CH)
    dst_w = dst.reshape(NW, NCH, CH)
    src_p = src.reshape(NW, NCHP, CHP)
    dst_p = dst.reshape(NW, NCHP, CHP)
    xb = x.astype(jnp.bfloat16)

    dp, bp = _degree(src_w, dst_w, HEW)
    dp = dp.reshape(NW, NN)
    bp = bp.reshape(NW, NE)

    ep = _phase(xb, src_p, dst_p)

    ef = pl.pallas_call(
        _combine_edges_body,
        out_shape=jax.ShapeDtypeStruct((NE, C), jnp.bfloat16),
    )(ep, bp)

    npar = _phase(ef, dst_p, src_p)

    out = pl.pallas_call(
        _finish_body,
        out_shape=jax.ShapeDtypeStruct((NN, C), jnp.float32),
    )(npar, dp, W, b.reshape(1, C))
    return out


# speedup vs baseline: 1.4794x; 1.0004x over previous
import jax
import jax.numpy as jnp
from jax import lax
from jax.experimental import pallas as pl
from jax.experimental.pallas import tpu as pltpu
from jax.experimental.pallas import tpu_sc as plsc

NN = 10000
NE = 10000
NNZ = 320000
C = 128
NC = 2
NS = 16
NW = NC * NS
EPW = NNZ // NW
CH = 80
NCH = EPW // CH
CHP = 125
NCHP = EPW // CHP
ACC_ROWS = 10240
ZR = ACC_ROWS // NS


def _degree_body(gi_hbm, si_hbm, hew_hbm, dp_out, bp_out,
                 gi_v, si_v, hew_v, d_v, b_v):
    cid = lax.axis_index("c")
    sid = lax.axis_index("s")
    wid = sid * NC + cid

    pltpu.sync_copy(gi_hbm.at[wid], gi_v)
    pltpu.sync_copy(si_hbm.at[wid], si_v)
    pltpu.sync_copy(hew_hbm, hew_v)

    z16 = jnp.zeros((16,), jnp.float32)

    def zdb(i, carry):
        d_v[pl.ds(i * 16, 16)] = z16
        b_v[pl.ds(i * 16, 16)] = z16
        return carry

    lax.fori_loop(0, NN // 16, zdb, 0)

    ones16 = jnp.ones((16,), jnp.float32)

    def chunk(c, carry):
        for g in range(CH // 16):
            s16 = gi_v[c, pl.ds(g * 16, 16)]
            d16 = si_v[c, pl.ds(g * 16, 16)]
            w16 = plsc.load_gather(hew_v, [d16])
            plsc.addupdate_scatter(d_v, [s16], w16)
            plsc.addupdate_scatter(b_v, [d16], ones16)
        return carry

    lax.fori_loop(0, NCH, chunk, 0)

    obase = pl.multiple_of(wid * NN, 8)
    pltpu.sync_copy(d_v, dp_out.at[pl.ds(obase, NN)])
    pltpu.sync_copy(b_v, bp_out.at[pl.ds(obase, NN)])


NBUF = 8


def _phase_body(tab_hbm, gi_hbm, si_hbm, acc_out,
                gi_v, si_v, *bufs_and_sems):
    rows = bufs_and_sems[:NBUF]
    acc_sh = bufs_and_sems[NBUF]
    gs = bufs_and_sems[NBUF + 1:2 * NBUF + 1]
    ss = bufs_and_sems[2 * NBUF + 1:]
    cid = lax.axis_index("c")
    sid = lax.axis_index("s")
    wid = sid * NC + cid

    pltpu.sync_copy(gi_hbm.at[wid], gi_v)
    pltpu.sync_copy(si_hbm.at[wid], si_v)

    z32 = jnp.zeros((32,), jnp.bfloat16)

    def zrow(i, carry):
        rows[0][i // 4, pl.ds((i % 4) * 32, 32)] = z32
        return carry

    lax.fori_loop(0, CHP * 4, zrow, 0)
    zbase = pl.multiple_of(sid * ZR, 8)
    for k in range(ZR // CH):
        pltpu.sync_copy(rows[0].at[pl.ds(0, CH)],
                        acc_sh.at[pl.ds(zbase + k * CH, CH)])
    plsc.subcore_barrier()

    def gather(c, rows_v, sem):
        pltpu.async_copy(tab_hbm.at[gi_v.at[c]], rows_v, sem)

    def gather_wait(c, rows_v, sem):
        pltpu.make_async_copy(tab_hbm.at[gi_v.at[c]], rows_v, sem).wait()

    def scatter(c, rows_v, sem):
        pltpu.async_copy(rows_v, acc_sh.at[si_v.at[c]], sem, add=True)

    def scatter_wait(c, rows_v, sem):
        pltpu.make_async_copy(rows_v, acc_sh.at[si_v.at[c]], sem).wait()

    for j in range(NBUF):
        gather(j, rows[j], gs[j])

    def block(i, carry):
        c0 = i * NBUF
        for j in range(NBUF):
            gather_wait(c0 + j, rows[j], gs[j])
            scatter(c0 + j, rows[j], ss[j])
        for j in range(NBUF):
            cn = c0 + NBUF + j

            @pl.when(cn < NCHP)
            def _(j=j, cn=cn):
                scatter_wait(c0 + j, rows[j], ss[j])
                gather(cn, rows[j], gs[j])

        return carry

    lax.fori_loop(0, NCHP // NBUF, block, 0)
    for j in range(NBUF):
        scatter_wait(NCHP - NBUF + j, rows[j], ss[j])
    plsc.subcore_barrier()

    pltpu.sync_copy(acc_sh.at[pl.ds(zbase, ZR)],
                    acc_out.at[cid, pl.ds(zbase, ZR)])


_MESH = plsc.VectorSubcoreMesh(core_axis_name="c", subcore_axis_name="s",
                               num_cores=NC, num_subcores=NS)
_SC_PARAMS = pltpu.CompilerParams(needs_layout_passes=False,
                                  use_tc_tiling_on_sc=False)

_degree = pl.kernel(
    _degree_body,
    out_type=(
        jax.ShapeDtypeStruct((NW * NN,), jnp.float32),
        jax.ShapeDtypeStruct((NW * NE,), jnp.float32),
    ),
    mesh=_MESH,
    compiler_params=_SC_PARAMS,
    scratch_types=(
        pltpu.VMEM((NCH, CH), jnp.int32),
        pltpu.VMEM((NCH, CH), jnp.int32),
        pltpu.VMEM((NE,), jnp.float32),
        pltpu.VMEM((NN,), jnp.float32),
        pltpu.VMEM((NE,), jnp.float32),
    ),
)

_phase = pl.kernel(
    _phase_body,
    out_type=jax.ShapeDtypeStruct((NC, ACC_ROWS, C), jnp.bfloat16),
    mesh=_MESH,
    compiler_params=_SC_PARAMS,
    scratch_types=(
        (pltpu.VMEM((NCHP, CHP), jnp.int32),
         pltpu.VMEM((NCHP, CHP), jnp.int32))
        + tuple(pltpu.VMEM((CHP, C), jnp.bfloat16) for _ in range(NBUF))
        + (pltpu.VMEM_SHARED((ACC_ROWS, C), jnp.bfloat16),)
        + tuple(pltpu.SemaphoreType.DMA for _ in range(2 * NBUF))
    ),
)


def _combine_edges_body(ep_ref, bp_ref, out_ref):
    bsum = jnp.sum(bp_ref[...], axis=0)
    binv = jnp.where(bsum > 0, 1.0 / bsum, 0.0)
    esum = (ep_ref[0, :NE].astype(jnp.float32)
            + ep_ref[1, :NE].astype(jnp.float32))
    out_ref[...] = (esum * binv[:, None]).astype(jnp.bfloat16)


def _finish_body(np_ref, dp_ref, w_ref, b_ref, out_ref):
    dsum = jnp.sum(dp_ref[...], axis=0)
    dinv = jnp.where(dsum > 0, 1.0 / dsum, 0.0)
    t = (np_ref[0, :NN].astype(jnp.float32)
         + np_ref[1, :NN].astype(jnp.float32)) * dinv[:, None]
    out_ref[...] = (jnp.dot(t, w_ref[...], preferred_element_type=jnp.float32)
                    + b_ref[...])


def kernel(x, HE, HEW, W, b):
    src = HE[0]
    dst = HE[1]
    src_w = src.reshape(NW, NCH, CH)
    dst_w = dst.reshape(NW, NCH, CH)
    src_p = src.reshape(NW, NCHP, CHP)
    dst_p = dst.reshape(NW, NCHP, CHP)
    xb = x.astype(jnp.bfloat16)

    dp, bp = _degree(src_w, dst_w, HEW)
    dp = dp.reshape(NW, NN)
    bp = bp.reshape(NW, NE)

    ep = _phase(xb, src_p, dst_p)

    ef = pl.pallas_call(
        _combine_edges_body,
        out_shape=jax.ShapeDtypeStruct((NE, C), jnp.bfloat16),
    )(ep, bp)

    npar = _phase(ef, dst_p, src_p)

    out = pl.pallas_call(
        _finish_body,
        out_shape=jax.ShapeDtypeStruct((NN, C), jnp.float32),
    )(npar, dp, W, b.reshape(1, C))
    return out
